# speedup vs baseline: 1.6402x; 1.6402x over previous
import functools

import jax
import jax.numpy as jnp
from jax.experimental import pallas as pl
from jax.experimental.pallas import tpu as pltpu

_E = 8
_HB = 1024


def _router_kernel(x_ref, wr_ref, g_ref):
    x = x_ref[...]
    logits = jnp.dot(x, wr_ref[...], preferred_element_type=jnp.float32)
    m = jnp.max(logits, axis=-1, keepdims=True)
    ex = jnp.exp(logits - m)
    probs = ex / jnp.sum(ex, axis=-1, keepdims=True)
    lane = jax.lax.broadcasted_iota(jnp.int32, probs.shape, 1)
    i1 = jnp.argmax(probs, axis=-1)[:, None]
    top1 = lane == i1
    masked = jnp.where(top1, -1.0, probs)
    i2 = jnp.argmax(masked, axis=-1)[:, None]
    top2 = lane == i2
    g_ref[...] = jnp.where(top1 | top2, probs, 0.0)


def _ffn_kernel(x_ref, g_ref, wv_ref, w_ref, w1_ref, o_ref, acc_ref, *, nh):
    e = pl.program_id(0)
    h = pl.program_id(1)

    @pl.when((e == 0) & (h == 0))
    def _init():
        acc_ref[...] = jnp.zeros_like(acc_ref)

    x = x_ref[...].astype(jnp.bfloat16)
    wv = wv_ref[0].astype(jnp.bfloat16)
    w = w_ref[0].astype(jnp.bfloat16)
    w1 = w1_ref[0].astype(jnp.bfloat16)
    v = jnp.dot(x, wv, preferred_element_type=jnp.float32)
    g = jax.nn.gelu(jnp.dot(x, w, preferred_element_type=jnp.float32))
    p = (v * g).astype(jnp.bfloat16)
    y = jnp.dot(p, w1, preferred_element_type=jnp.float32)
    lane = jax.lax.broadcasted_iota(jnp.int32, g_ref.shape, 1)
    gate = jnp.sum(jnp.where(lane == e, g_ref[...], 0.0), axis=1, keepdims=True)
    acc_ref[...] += y * gate

    @pl.when((e == _E - 1) & (h == nh - 1))
    def _fin():
        o_ref[...] = acc_ref[...].astype(jnp.bfloat16)


@jax.jit
def kernel(inputs, padding_mask, w_router, w_v, w, w1):
    B, S, D = inputs.shape
    T = B * S
    H = w_v.shape[2]
    nh = H // _HB
    x = inputs.reshape(T, D)

    gates = pl.pallas_call(
        _router_kernel,
        out_shape=jax.ShapeDtypeStruct((T, _E), jnp.float32),
    )(x, w_router)

    out = pl.pallas_call(
        functools.partial(_ffn_kernel, nh=nh),
        grid=(_E, nh),
        in_specs=[
            pl.BlockSpec((T, D), lambda e, h: (0, 0)),
            pl.BlockSpec((T, _E), lambda e, h: (0, 0)),
            pl.BlockSpec((1, D, _HB), lambda e, h: (e, 0, h)),
            pl.BlockSpec((1, D, _HB), lambda e, h: (e, 0, h)),
            pl.BlockSpec((1, _HB, D), lambda e, h: (e, h, 0)),
        ],
        out_specs=pl.BlockSpec((T, D), lambda e, h: (0, 0)),
        out_shape=jax.ShapeDtypeStruct((T, D), jnp.bfloat16),
        scratch_shapes=[pltpu.VMEM((T, D), jnp.float32)],
    )(x, gates, w_v, w, w1)
    return out.reshape(B, S, D)
